# Initial kernel scaffold; baseline (speedup 1.0000x reference)
import jax
import jax.numpy as jnp
from jax.experimental import pallas as pl


def kernel(q, k, q_pos_emb, k_pos_emb, linear_sort_q, linear_sort_k):
    raise NotImplementedError("write your pallas kernel here")


# speedup vs baseline: 1.1131x; 1.1131x over previous
import jax
import jax.numpy as jnp
from jax import lax
from jax.experimental import pallas as pl

HEADS = 16
BUCKETS = 64
SEQ = 4096
DIM = 128
TOK_PER_BUCKET = SEQ // BUCKETS


def _body(q_ref, k_ref, qpos_ref, kpos_ref, wq_ref, wk_ref, out_ref):
    qb = q_ref[0]
    kb = k_ref[0]

    mq = jnp.sum(qb.reshape(BUCKETS, TOK_PER_BUCKET, DIM), axis=1) * (
        jnp.float32(1.0 / TOK_PER_BUCKET))
    mk = jnp.sum(kb.reshape(BUCKETS, TOK_PER_BUCKET, DIM), axis=1) * (
        jnp.float32(1.0 / TOK_PER_BUCKET))

    wq = wq_ref[0, 0]
    wk = wk_ref[0, 0]
    hi = None
    sq = (jnp.dot(mq, wq[:DIM], preferred_element_type=jnp.float32, precision=hi)
          + jnp.dot(qpos_ref[0, 0], wq[DIM:],
                    preferred_element_type=jnp.float32, precision=hi))
    sk = (jnp.dot(mk, wk[:DIM], preferred_element_type=jnp.float32, precision=hi)
          + jnp.dot(kpos_ref[0, 0], wk[DIM:],
                    preferred_element_type=jnp.float32, precision=hi))

    r = lax.dot_general(sq, sk, (((1,), (1,)), ((), ())),
                        preferred_element_type=jnp.float32,
                        precision=hi)
    r = r - jnp.max(r, axis=-1, keepdims=True)
    e = jnp.exp(r)
    out_ref[0] = e / jnp.sum(e, axis=-1, keepdims=True)


def kernel(q, k, q_pos_emb, k_pos_emb, linear_sort_q, linear_sort_k):
    bh = q.shape[0]
    return pl.pallas_call(
        _body,
        grid=(bh,),
        in_specs=[
            pl.BlockSpec((1, SEQ, DIM), lambda i: (i, 0, 0)),
            pl.BlockSpec((1, SEQ, DIM), lambda i: (i, 0, 0)),
            pl.BlockSpec((1, 1, BUCKETS, DIM), lambda i: (0, i % HEADS, 0, 0)),
            pl.BlockSpec((1, 1, BUCKETS, DIM), lambda i: (0, i % HEADS, 0, 0)),
            pl.BlockSpec((1, 1, 2 * DIM, DIM), lambda i: (0, i % HEADS, 0, 0)),
            pl.BlockSpec((1, 1, 2 * DIM, DIM), lambda i: (0, i % HEADS, 0, 0)),
        ],
        out_specs=pl.BlockSpec((1, BUCKETS, BUCKETS), lambda i: (i, 0, 0)),
        out_shape=jax.ShapeDtypeStruct((bh, BUCKETS, BUCKETS), jnp.float32),
    )(q, k, q_pos_emb, k_pos_emb, linear_sort_q, linear_sort_k)
